# baseline (speedup 1.0000x reference)
import functools

import jax
import jax.numpy as jnp
from jax import lax
from jax.experimental import pallas as pl
from jax.experimental.pallas import tpu as pltpu
from jax.experimental.pallas import tpu_sc as plsc

NN = 4096
DD = 768
KTOP = 5
RB = 256

NSC = 2
NTILE = 16
TROWS = NN // NTILE
PPR = 2 * KTOP + 1
NPAIR = TROWS * PPR
CH = 64
NCH = NPAIR // CH
FC = 128
NF = 3


def _topk_body(a_ref, ti_ref, v_ref):
    blk = a_ref[...]
    cols = lax.broadcasted_iota(jnp.int32, blk.shape, 1)
    cur = blk
    vs, idxs = [], []
    for _ in range(KTOP):
        m = jnp.max(cur, axis=1, keepdims=True)
        idx = jnp.min(jnp.where(cur == m, cols, blk.shape[1]), axis=1,
                      keepdims=True)
        vs.append(m)
        idxs.append(idx)
        cur = jnp.where(cols == idx, -jnp.inf, cur)
    v5 = jnp.concatenate(vs, axis=1)
    i5 = jnp.concatenate(idxs, axis=1)
    t = v5[:, KTOP - 1:KTOP]
    c = jnp.max(jnp.where(v5 == t, i5, -1), axis=1, keepdims=True)
    ipad = jnp.zeros((blk.shape[0], 2), jnp.int32)
    fpad = jnp.zeros((blk.shape[0], 3), jnp.float32)
    ti_ref[...] = jnp.concatenate([i5, c, ipad], axis=1)
    v_ref[...] = jnp.concatenate([v5, fpad], axis=1)


def _run_topk(A):
    return pl.pallas_call(
        _topk_body,
        grid=(NN // RB,),
        in_specs=[pl.BlockSpec((RB, NN), lambda i: (i, 0))],
        out_specs=[pl.BlockSpec((RB, 8), lambda i: (i, 0)),
                   pl.BlockSpec((RB, 8), lambda i: (i, 0))],
        out_shape=[jax.ShapeDtypeStruct((NN, 8), jnp.int32),
                   jax.ShapeDtypeStruct((NN, 8), jnp.float32)],
    )(A)


def _ws_body(a_ref, ti_ref, v_ref, t2_ref, c2_ref, w_ref):
    i = pl.program_id(0)
    blk = a_ref[...]
    ti8 = ti_ref[...]
    v8 = v_ref[...]
    t2 = t2_ref[...]
    c2 = c2_ref[...]
    cols = lax.broadcasted_iota(jnp.int32, blk.shape, 1)
    rows = i * RB + lax.broadcasted_iota(jnp.int32, blk.shape, 0)
    t_i = v8[:, KTOP - 1:KTOP]
    c_i = ti8[:, KTOP:KTOP + 1]
    inrow = (blk > t_i) | ((blk == t_i) & (cols <= c_i))
    incol = (blk > t2) | ((blk == t2) & (rows <= c2))
    s = 1.0 + jnp.sum(jnp.where(inrow | incol, blk, 0.0), axis=1,
                      keepdims=True)
    wcols = []
    for k in range(KTOP):
        tik = ti8[:, k:k + 1]
        sel = cols == tik
        tj = jnp.max(jnp.where(sel, t2, -jnp.inf), axis=1, keepdims=True)
        cj = jnp.max(jnp.where(sel, c2, -1), axis=1, keepdims=True)
        vk = v8[:, k:k + 1]
        mut = (vk > tj) | ((vk == tj) & (rows[:, :1] <= cj))
        wcols.append(vk * jnp.where(mut, 0.5, 1.0))
    fpad = jnp.zeros((blk.shape[0], 2), jnp.float32)
    w_ref[...] = jnp.concatenate(wcols + [s, fpad], axis=1)


def _run_ws(A, ti8, v8):
    t2 = v8[:, KTOP - 1].reshape(1, NN)
    c2 = ti8[:, KTOP].reshape(1, NN)
    return pl.pallas_call(
        _ws_body,
        grid=(NN // RB,),
        in_specs=[pl.BlockSpec((RB, NN), lambda i: (i, 0)),
                  pl.BlockSpec((RB, 8), lambda i: (i, 0)),
                  pl.BlockSpec((RB, 8), lambda i: (i, 0)),
                  pl.BlockSpec((1, NN), lambda i: (0, 0)),
                  pl.BlockSpec((1, NN), lambda i: (0, 0))],
        out_specs=pl.BlockSpec((RB, 8), lambda i: (i, 0)),
        out_shape=jax.ShapeDtypeStruct((NN, 8), jnp.float32),
    )(A, ti8, v8, t2, c2)


def _sc_body(dst_hbm, src_hbm, w_hbm, table_hbm, z_hbm, out_hbm,
             dst_v, src_v, w_v, buf0, acc_sh, sem0):
    cid = lax.axis_index("c")
    sid = lax.axis_index("s")
    row0 = sid * TROWS
    lanes = lax.iota(jnp.int32, 16)

    pltpu.sync_copy(dst_hbm.at[sid], dst_v)
    pltpu.sync_copy(w_hbm.at[sid], w_v)

    for f in range(NF):
        pltpu.sync_copy(src_hbm.at[cid, f, sid], src_v)
        pltpu.sync_copy(z_hbm, acc_sh.at[pl.ds(row0, TROWS)])
        plsc.subcore_barrier()

        def process(ch, carry):
            pltpu.async_copy(table_hbm.at[src_v.at[ch]], buf0, sem0).wait()

            def grp(g, carry2):
                rows16 = g * 16 + lanes
                w16 = plsc.load_gather(
                    w_v, [jnp.full((16,), ch, jnp.int32), rows16])

                def colloop(cc, carry3):
                    c16 = jnp.full((16,), cc, jnp.int32)
                    x = plsc.load_gather(buf0, [rows16, c16])
                    plsc.store_scatter(buf0, [rows16, c16], x * w16)
                    return carry3

                lax.fori_loop(0, FC, colloop, 0)
                return carry2

            lax.fori_loop(0, CH // 16, grp, 0)
            pltpu.sync_copy(buf0, acc_sh.at[dst_v.at[ch]], add=True)
            return carry

        lax.fori_loop(0, NCH, process, 0)
        plsc.subcore_barrier()
        pltpu.sync_copy(acc_sh.at[pl.ds(row0, TROWS)],
                        out_hbm.at[cid * NF + f, pl.ds(row0, TROWS)])
        plsc.subcore_barrier()


def _run_sc(dst_in, src_in, w_in, table, zrows):
    mesh = plsc.VectorSubcoreMesh(core_axis_name="c", subcore_axis_name="s")
    f = functools.partial(
        pl.kernel,
        out_type=jax.ShapeDtypeStruct((NSC * NF, NN, FC), jnp.float32),
        mesh=mesh,
        compiler_params=pltpu.CompilerParams(needs_layout_passes=False),
        scratch_types=[
            pltpu.VMEM((NCH, CH), jnp.int32),
            pltpu.VMEM((NCH, CH), jnp.int32),
            pltpu.VMEM((NCH, CH), jnp.float32),
            pltpu.VMEM((CH, FC), jnp.float32),
            pltpu.VMEM_SHARED((NN, FC), jnp.float32),
            pltpu.SemaphoreType.DMA,
        ],
    )(_sc_body)
    return f(dst_in, src_in, w_in, table, zrows)


def _gelu_exact(x):
    return 0.5 * x * (1.0 + lax.erf(x * 0.7071067811865476))


def _tail_body(y_ref, s_ref, emb_ref, wg_ref, bg_ref, w1_ref, b1_ref,
               w2_ref, b2_ref, wh_ref, bh_ref, out_ref):
    hp = jax.lax.Precision.HIGHEST
    s = jnp.maximum(s_ref[...][:, KTOP:KTOP + 1], 1e-8)
    y = y_ref[...] / s
    emb = emb_ref[...]
    h = _gelu_exact(
        jnp.dot(y, wg_ref[...], preferred_element_type=jnp.float32,
                precision=hp) + bg_ref[...]) + emb
    h1 = _gelu_exact(
        jnp.dot(h, w1_ref[...], preferred_element_type=jnp.float32,
                precision=hp) + b1_ref[...])
    h2 = _gelu_exact(
        jnp.dot(h1, w2_ref[...], preferred_element_type=jnp.float32,
                precision=hp) + b2_ref[...])
    r = jnp.dot(h2, wh_ref[...], preferred_element_type=jnp.float32,
                precision=hp) + bh_ref[...]
    sp = jax.nn.softplus(r)
    col = lax.broadcasted_iota(jnp.int32, r.shape, 1)
    out8 = jnp.where(col == 0, r,
                     jnp.where(col == 2, jnp.minimum(sp, 28.0) + 1.01,
                               sp + 1e-6))
    out_ref[...] = out8[:, :4]


def _run_tail(ymsg, w8, emb, Wg_t, bg2, W1_t, b12, W2_t, b22, Wh_t, bh2):
    whole = lambda shape: pl.BlockSpec(shape, lambda i: tuple(0 for _ in shape))
    return pl.pallas_call(
        _tail_body,
        grid=(NN // RB,),
        in_specs=[
            pl.BlockSpec((RB, DD), lambda i: (i, 0)),
            pl.BlockSpec((RB, 8), lambda i: (i, 0)),
            pl.BlockSpec((RB, DD), lambda i: (i, 0)),
            whole((DD, DD)), whole((1, DD)),
            whole((DD, 512)), whole((1, 512)),
            whole((512, 256)), whole((1, 256)),
            whole((256, 8)), whole((1, 8)),
        ],
        out_specs=pl.BlockSpec((RB, 4), lambda i: (i, 0)),
        out_shape=jax.ShapeDtypeStruct((NN, 4), jnp.float32),
    )(ymsg, w8, emb, Wg_t, bg2, W1_t, b12, W2_t, b22, Wh_t, bh2)


def kernel(emb, A, W_gcn, b_gcn, W1, b1, W2, b2, Wh, bh):
    ti8, v8 = _run_topk(A)
    w8 = _run_ws(A, ti8, v8)

    rows = jnp.arange(NN, dtype=jnp.int32)[:, None]
    rows5 = jnp.broadcast_to(rows, (NN, KTOP))
    ti5 = ti8[:, :KTOP]
    w5 = w8[:, :KTOP]
    dst_flat = jnp.concatenate([rows5, ti5, rows], axis=1).reshape(-1)
    src_flat = jnp.concatenate([ti5, rows5, rows], axis=1).reshape(-1)
    w_flat = jnp.concatenate(
        [w5, w5, jnp.ones((NN, 1), jnp.float32)], axis=1).reshape(-1)
    dst_in = dst_flat.reshape(NTILE, NCH, CH)
    w_in = w_flat.reshape(NTILE, NCH, CH)
    qoff = (jnp.arange(NSC * NF, dtype=jnp.int32) * NN).reshape(
        NSC, NF, 1, 1, 1)
    src_in = src_flat.reshape(1, 1, NTILE, NCH, CH) + qoff
    table = emb.reshape(NN, NSC * NF, FC).transpose(1, 0, 2).reshape(
        NSC * NF * NN, FC)
    zrows = jnp.zeros((TROWS, FC), jnp.float32)

    out4 = _run_sc(dst_in, src_in, w_in, table, zrows)
    ymsg = out4.transpose(1, 0, 2).reshape(NN, DD)

    Wg_t = W_gcn.T
    W1_t = W1.T
    W2_t = W2.T
    Wh_t = jnp.pad(Wh, ((0, 4), (0, 0))).T
    bh2 = jnp.pad(bh, (0, 4)).reshape(1, 8)
    return _run_tail(ymsg, w8, emb, Wg_t, b_gcn.reshape(1, DD), W1_t,
                     b1.reshape(1, 512), W2_t, b2.reshape(1, 256), Wh_t, bh2)


# speedup vs baseline: 3.2886x; 3.2886x over previous
import functools

import jax
import jax.numpy as jnp
from jax import lax
from jax.experimental import pallas as pl
from jax.experimental.pallas import tpu as pltpu
from jax.experimental.pallas import tpu_sc as plsc

NN = 4096
DD = 768
KTOP = 5
RB = 256

NSC = 2
NTILE = 16
TROWS = NN // NTILE
PPR = 2 * KTOP + 1
NPAIR = TROWS * PPR
CH = 64
NCH = NPAIR // CH
FC = 128
NF = 3


def _topk_body(a_ref, ti_ref, v_ref):
    blk = a_ref[...]
    cols = lax.broadcasted_iota(jnp.int32, blk.shape, 1)
    cur = blk
    vs, idxs = [], []
    for _ in range(KTOP):
        m = jnp.max(cur, axis=1, keepdims=True)
        idx = jnp.min(jnp.where(cur == m, cols, blk.shape[1]), axis=1,
                      keepdims=True)
        vs.append(m)
        idxs.append(idx)
        cur = jnp.where(cols == idx, -jnp.inf, cur)
    v5 = jnp.concatenate(vs, axis=1)
    i5 = jnp.concatenate(idxs, axis=1)
    t = v5[:, KTOP - 1:KTOP]
    c = jnp.max(jnp.where(v5 == t, i5, -1), axis=1, keepdims=True)
    ipad = jnp.zeros((blk.shape[0], 2), jnp.int32)
    fpad = jnp.zeros((blk.shape[0], 3), jnp.float32)
    ti_ref[...] = jnp.concatenate([i5, c, ipad], axis=1)
    v_ref[...] = jnp.concatenate([v5, fpad], axis=1)


def _run_topk(A):
    return pl.pallas_call(
        _topk_body,
        grid=(NN // RB,),
        in_specs=[pl.BlockSpec((RB, NN), lambda i: (i, 0))],
        out_specs=[pl.BlockSpec((RB, 8), lambda i: (i, 0)),
                   pl.BlockSpec((RB, 8), lambda i: (i, 0))],
        out_shape=[jax.ShapeDtypeStruct((NN, 8), jnp.int32),
                   jax.ShapeDtypeStruct((NN, 8), jnp.float32)],
    )(A)


def _ws_body(a_ref, ti_ref, v_ref, t2_ref, c2_ref, w_ref):
    i = pl.program_id(0)
    blk = a_ref[...]
    ti8 = ti_ref[...]
    v8 = v_ref[...]
    t2 = t2_ref[...]
    c2 = c2_ref[...]
    cols = lax.broadcasted_iota(jnp.int32, blk.shape, 1)
    rows = i * RB + lax.broadcasted_iota(jnp.int32, blk.shape, 0)
    t_i = v8[:, KTOP - 1:KTOP]
    c_i = ti8[:, KTOP:KTOP + 1]
    inrow = (blk > t_i) | ((blk == t_i) & (cols <= c_i))
    incol = (blk > t2) | ((blk == t2) & (rows <= c2))
    s = 1.0 + jnp.sum(jnp.where(inrow | incol, blk, 0.0), axis=1,
                      keepdims=True)
    wcols = []
    for k in range(KTOP):
        tik = ti8[:, k:k + 1]
        sel = cols == tik
        tj = jnp.max(jnp.where(sel, t2, -jnp.inf), axis=1, keepdims=True)
        cj = jnp.max(jnp.where(sel, c2, -1), axis=1, keepdims=True)
        vk = v8[:, k:k + 1]
        mut = (vk > tj) | ((vk == tj) & (rows[:, :1] <= cj))
        wcols.append(vk * jnp.where(mut, 0.5, 1.0))
    fpad = jnp.zeros((blk.shape[0], 2), jnp.float32)
    w_ref[...] = jnp.concatenate(wcols + [s, fpad], axis=1)


def _run_ws(A, ti8, v8):
    t2 = v8[:, KTOP - 1].reshape(1, NN)
    c2 = ti8[:, KTOP].reshape(1, NN)
    return pl.pallas_call(
        _ws_body,
        grid=(NN // RB,),
        in_specs=[pl.BlockSpec((RB, NN), lambda i: (i, 0)),
                  pl.BlockSpec((RB, 8), lambda i: (i, 0)),
                  pl.BlockSpec((RB, 8), lambda i: (i, 0)),
                  pl.BlockSpec((1, NN), lambda i: (0, 0)),
                  pl.BlockSpec((1, NN), lambda i: (0, 0))],
        out_specs=pl.BlockSpec((RB, 8), lambda i: (i, 0)),
        out_shape=jax.ShapeDtypeStruct((NN, 8), jnp.float32),
    )(A, ti8, v8, t2, c2)


def _sc_body(dst_hbm, src_hbm, w_hbm, table_hbm, z_hbm, out_hbm,
             dst_v, src_v, w_v, buf0, buf1, acc_sh, sem0, sem1):
    cid = lax.axis_index("c")
    sid = lax.axis_index("s")
    row0 = sid * TROWS
    lanes = lax.iota(jnp.int32, 16)
    bufs = (buf0, buf1)
    sems = (sem0, sem1)

    pltpu.sync_copy(dst_hbm.at[sid], dst_v)
    pltpu.sync_copy(w_hbm.at[sid], w_v)

    for f in range(NF):
        pltpu.sync_copy(src_hbm.at[cid, f, sid], src_v)
        pltpu.sync_copy(z_hbm, acc_sh.at[pl.ds(row0, TROWS)])
        plsc.subcore_barrier()

        pltpu.async_copy(table_hbm.at[src_v.at[0]], buf0, sem0)

        def process2(i2, carry):
            for par in range(2):
                ch = i2 * 2 + par
                buf = bufs[par]
                pltpu.make_async_copy(table_hbm.at[src_v.at[ch]], buf,
                                      sems[par]).wait()

                @pl.when(ch + 1 < NCH)
                def _():
                    pltpu.async_copy(table_hbm.at[src_v.at[ch + 1]],
                                     bufs[1 - par], sems[1 - par])

                def grp(g, carry2):
                    rows16 = g * 16 + lanes
                    w16 = plsc.load_gather(
                        w_v, [jnp.full((16,), ch, jnp.int32), rows16])

                    @functools.partial(plsc.parallel_loop, 0, FC,
                                       unroll=8)
                    def colloop(cc):
                        c16 = jnp.full((16,), cc, jnp.int32)
                        x = plsc.load_gather(buf, [rows16, c16])
                        plsc.store_scatter(buf, [rows16, c16], x * w16)

                    return carry2

                lax.fori_loop(0, CH // 16, grp, 0)
                pltpu.sync_copy(buf, acc_sh.at[dst_v.at[ch]], add=True)
            return carry

        lax.fori_loop(0, NCH // 2, process2, 0)
        plsc.subcore_barrier()
        pltpu.sync_copy(acc_sh.at[pl.ds(row0, TROWS)],
                        out_hbm.at[cid * NF + f, pl.ds(row0, TROWS)])
        plsc.subcore_barrier()


def _run_sc(dst_in, src_in, w_in, table, zrows):
    mesh = plsc.VectorSubcoreMesh(core_axis_name="c", subcore_axis_name="s")
    f = functools.partial(
        pl.kernel,
        out_type=jax.ShapeDtypeStruct((NSC * NF, NN, FC), jnp.float32),
        mesh=mesh,
        compiler_params=pltpu.CompilerParams(needs_layout_passes=False),
        scratch_types=[
            pltpu.VMEM((NCH, CH), jnp.int32),
            pltpu.VMEM((NCH, CH), jnp.int32),
            pltpu.VMEM((NCH, CH), jnp.float32),
            pltpu.VMEM((CH, FC), jnp.float32),
            pltpu.VMEM((CH, FC), jnp.float32),
            pltpu.VMEM_SHARED((NN, FC), jnp.float32),
            pltpu.SemaphoreType.DMA,
            pltpu.SemaphoreType.DMA,
        ],
    )(_sc_body)
    return f(dst_in, src_in, w_in, table, zrows)


def _gelu_exact(x):
    return 0.5 * x * (1.0 + lax.erf(x * 0.7071067811865476))


def _tail_body(y_ref, s_ref, emb_ref, wg_ref, bg_ref, w1_ref, b1_ref,
               w2_ref, b2_ref, wh_ref, bh_ref, out_ref):
    hp = jax.lax.Precision.HIGHEST
    s = jnp.maximum(s_ref[...][:, KTOP:KTOP + 1], 1e-8)
    y = y_ref[...] / s
    emb = emb_ref[...]
    h = _gelu_exact(
        jnp.dot(y, wg_ref[...], preferred_element_type=jnp.float32,
                precision=hp) + bg_ref[...]) + emb
    h1 = _gelu_exact(
        jnp.dot(h, w1_ref[...], preferred_element_type=jnp.float32,
                precision=hp) + b1_ref[...])
    h2 = _gelu_exact(
        jnp.dot(h1, w2_ref[...], preferred_element_type=jnp.float32,
                precision=hp) + b2_ref[...])
    r = jnp.dot(h2, wh_ref[...], preferred_element_type=jnp.float32,
                precision=hp) + bh_ref[...]
    sp = jax.nn.softplus(r)
    col = lax.broadcasted_iota(jnp.int32, r.shape, 1)
    out8 = jnp.where(col == 0, r,
                     jnp.where(col == 2, jnp.minimum(sp, 28.0) + 1.01,
                               sp + 1e-6))
    out_ref[...] = out8[:, :4]


def _run_tail(ymsg, w8, emb, Wg_t, bg2, W1_t, b12, W2_t, b22, Wh_t, bh2):
    whole = lambda shape: pl.BlockSpec(shape, lambda i: tuple(0 for _ in shape))
    return pl.pallas_call(
        _tail_body,
        grid=(NN // RB,),
        in_specs=[
            pl.BlockSpec((RB, DD), lambda i: (i, 0)),
            pl.BlockSpec((RB, 8), lambda i: (i, 0)),
            pl.BlockSpec((RB, DD), lambda i: (i, 0)),
            whole((DD, DD)), whole((1, DD)),
            whole((DD, 512)), whole((1, 512)),
            whole((512, 256)), whole((1, 256)),
            whole((256, 8)), whole((1, 8)),
        ],
        out_specs=pl.BlockSpec((RB, 4), lambda i: (i, 0)),
        out_shape=jax.ShapeDtypeStruct((NN, 4), jnp.float32),
    )(ymsg, w8, emb, Wg_t, bg2, W1_t, b12, W2_t, b22, Wh_t, bh2)


def kernel(emb, A, W_gcn, b_gcn, W1, b1, W2, b2, Wh, bh):
    ti8, v8 = _run_topk(A)
    w8 = _run_ws(A, ti8, v8)

    rows = jnp.arange(NN, dtype=jnp.int32)[:, None]
    rows5 = jnp.broadcast_to(rows, (NN, KTOP))
    ti5 = ti8[:, :KTOP]
    w5 = w8[:, :KTOP]
    dst_flat = jnp.concatenate([rows5, ti5, rows], axis=1).reshape(-1)
    src_flat = jnp.concatenate([ti5, rows5, rows], axis=1).reshape(-1)
    w_flat = jnp.concatenate(
        [w5, w5, jnp.ones((NN, 1), jnp.float32)], axis=1).reshape(-1)
    dst_in = dst_flat.reshape(NTILE, NCH, CH)
    w_in = w_flat.reshape(NTILE, NCH, CH)
    qoff = (jnp.arange(NSC * NF, dtype=jnp.int32) * NN).reshape(
        NSC, NF, 1, 1, 1)
    src_in = src_flat.reshape(1, 1, NTILE, NCH, CH) + qoff
    table = emb.reshape(NN, NSC * NF, FC).transpose(1, 0, 2).reshape(
        NSC * NF * NN, FC)
    zrows = jnp.zeros((TROWS, FC), jnp.float32)

    out4 = _run_sc(dst_in, src_in, w_in, table, zrows)
    ymsg = out4.transpose(1, 0, 2).reshape(NN, DD)

    Wg_t = W_gcn.T
    W1_t = W1.T
    W2_t = W2.T
    Wh_t = jnp.pad(Wh, ((0, 4), (0, 0))).T
    bh2 = jnp.pad(bh, (0, 4)).reshape(1, 8)
    return _run_tail(ymsg, w8, emb, Wg_t, b_gcn.reshape(1, DD), W1_t,
                     b1.reshape(1, 512), W2_t, b2.reshape(1, 256), Wh_t, bh2)


# speedup vs baseline: 4.0637x; 1.2357x over previous
import functools

import jax
import jax.numpy as jnp
from jax import lax
from jax.experimental import pallas as pl
from jax.experimental.pallas import tpu as pltpu
from jax.experimental.pallas import tpu_sc as plsc

NN = 4096
DD = 768
KTOP = 5
RB = 256

NSC = 2
NTILE = 16
TROWS = NN // NTILE
RPC = 8
GW = RPC * (KTOP + 1)
GCH = TROWS // RPC
FC = 128
NF = 3


def _topk_body(a_ref, ti_ref, v_ref):
    blk = a_ref[...]
    cols = lax.broadcasted_iota(jnp.int32, blk.shape, 1)
    cur = blk
    vs, idxs = [], []
    for _ in range(KTOP):
        m = jnp.max(cur, axis=1, keepdims=True)
        idx = jnp.min(jnp.where(cur == m, cols, blk.shape[1]), axis=1,
                      keepdims=True)
        vs.append(m)
        idxs.append(idx)
        cur = jnp.where(cols == idx, -jnp.inf, cur)
    v5 = jnp.concatenate(vs, axis=1)
    i5 = jnp.concatenate(idxs, axis=1)
    t = v5[:, KTOP - 1:KTOP]
    c = jnp.max(jnp.where(v5 == t, i5, -1), axis=1, keepdims=True)
    ipad = jnp.zeros((blk.shape[0], 2), jnp.int32)
    fpad = jnp.zeros((blk.shape[0], 3), jnp.float32)
    ti_ref[...] = jnp.concatenate([i5, c, ipad], axis=1)
    v_ref[...] = jnp.concatenate([v5, fpad], axis=1)


def _run_topk(A):
    return pl.pallas_call(
        _topk_body,
        grid=(NN // RB,),
        in_specs=[pl.BlockSpec((RB, NN), lambda i: (i, 0))],
        out_specs=[pl.BlockSpec((RB, 8), lambda i: (i, 0)),
                   pl.BlockSpec((RB, 8), lambda i: (i, 0))],
        out_shape=[jax.ShapeDtypeStruct((NN, 8), jnp.int32),
                   jax.ShapeDtypeStruct((NN, 8), jnp.float32)],
    )(A)


def _ws_body(a_ref, ti_ref, v_ref, t2_ref, c2_ref, w_ref):
    i = pl.program_id(0)
    blk = a_ref[...]
    t2 = t2_ref[...]
    c2 = c2_ref[...]
    cols = lax.broadcasted_iota(jnp.int32, blk.shape, 1)
    rows = i * RB + lax.broadcasted_iota(jnp.int32, blk.shape, 0)
    t_i = v_ref[...][:, KTOP - 1:KTOP]
    c_i = ti_ref[...][:, KTOP:KTOP + 1]
    inrow = (blk > t_i) | ((blk == t_i) & (cols <= c_i))
    incol = (blk > t2) | ((blk == t2) & (rows <= c2))
    s = 1.0 + jnp.sum(jnp.where(inrow | incol, blk, 0.0), axis=1,
                      keepdims=True)
    ti8 = ti_ref[...]
    v8 = v_ref[...]
    wcols = []
    for k in range(KTOP):
        tik = ti8[:, k:k + 1]
        sel = cols == tik
        tj = jnp.max(jnp.where(sel, t2, -jnp.inf), axis=1, keepdims=True)
        cj = jnp.max(jnp.where(sel, c2, -1), axis=1, keepdims=True)
        vk = v8[:, k:k + 1]
        mut = (vk > tj) | ((vk == tj) & (rows[:, :1] <= cj))
        wcols.append(vk * jnp.where(mut, 0.5, 1.0))
    zp2 = jnp.zeros((blk.shape[0], 2), jnp.float32)
    w_ref[...] = jnp.concatenate(wcols + [s, zp2], axis=1)


def _run_ws(A, ti8, v8):
    t2 = v8[:, KTOP - 1].reshape(1, NN)
    c2 = ti8[:, KTOP].reshape(1, NN)
    return pl.pallas_call(
        _ws_body,
        grid=(NN // RB,),
        in_specs=[pl.BlockSpec((RB, NN), lambda i: (i, 0)),
                  pl.BlockSpec((RB, 8), lambda i: (i, 0)),
                  pl.BlockSpec((RB, 8), lambda i: (i, 0)),
                  pl.BlockSpec((1, NN), lambda i: (0, 0)),
                  pl.BlockSpec((1, NN), lambda i: (0, 0))],
        out_specs=pl.BlockSpec((RB, 8), lambda i: (i, 0)),
        out_shape=jax.ShapeDtypeStruct((NN, 8), jnp.float32),
    )(A, ti8, v8, t2, c2)


def _table_body(e_ref, t_ref):
    e = e_ref[...]
    t_ref[...] = e.reshape(RB, NSC * NF, FC).transpose(1, 0, 2)


def _run_table(emb):
    return pl.pallas_call(
        _table_body,
        grid=(NN // RB,),
        in_specs=[pl.BlockSpec((RB, DD), lambda i: (i, 0))],
        out_specs=pl.BlockSpec((NSC * NF, RB, FC), lambda i: (0, i, 0)),
        out_shape=jax.ShapeDtypeStruct((NSC * NF, NN, FC), jnp.float32),
    )(emb)


def _sc_body(glist_hbm, dlist_hbm, w_hbm, table_hbm, z_hbm, out_hbm,
             glist_v, dlist_v, w_v,
             gbuf0, gbuf1, obuf0, obuf1, acc_sh,
             gsem0, gsem1, ssem0, ssem1):
    cid = lax.axis_index("c")
    sid = lax.axis_index("s")
    row0 = sid * TROWS
    lanes = lax.iota(jnp.int32, 16)
    gbufs = (gbuf0, gbuf1)
    obufs = (obuf0, obuf1)
    gsems = (gsem0, gsem1)
    ssems = (ssem0, ssem1)

    pltpu.sync_copy(dlist_hbm.at[sid], dlist_v)
    pltpu.sync_copy(w_hbm.at[pl.ds(row0, TROWS)], w_v)

    for f in range(NF):
        pltpu.sync_copy(glist_hbm.at[cid, f, sid], glist_v)
        pltpu.sync_copy(z_hbm, acc_sh.at[pl.ds(row0, TROWS)])
        plsc.subcore_barrier()

        pltpu.async_copy(table_hbm.at[glist_v.at[0]], gbuf0, gsem0)

        def process2(i2, carry):
            for par in range(2):
                ch = i2 * 2 + par
                gbuf = gbufs[par]
                obuf = obufs[par]
                pltpu.make_async_copy(table_hbm.at[glist_v.at[ch]], gbuf,
                                      gsems[par]).wait()

                @pl.when(ch + 1 < GCH)
                def _():
                    pltpu.async_copy(table_hbm.at[glist_v.at[ch + 1]],
                                     gbufs[1 - par], gsems[1 - par])

                @pl.when(ch >= 2)
                def _():
                    pltpu.make_async_copy(obuf, acc_sh.at[dlist_v.at[ch - 2]],
                                          ssems[par]).wait()

                def rowgrp(r, carry2):
                    b = r * (KTOP + 1)
                    rloc = ch * RPC + r
                    rsp = jnp.full((16,), rloc, jnp.int32)
                    ws = [plsc.load_gather(
                        w_v, [rsp, jnp.full((16,), k, jnp.int32)])
                        for k in range(KTOP)]

                    @functools.partial(plsc.parallel_loop, 0, FC // 16,
                                       unroll=4)
                    def colq(q):
                        c16 = q * 16 + lanes
                        b16 = jnp.full((16,), b, jnp.int32)
                        g0 = plsc.load_gather(gbuf, [b16, c16])
                        acc = g0
                        for k in range(KTOP):
                            bk = jnp.full((16,), b + 1 + k, jnp.int32)
                            gk = plsc.load_gather(gbuf, [bk, c16])
                            acc = acc + ws[k] * gk
                            plsc.store_scatter(obuf, [bk, c16], ws[k] * g0)
                        plsc.store_scatter(obuf, [b16, c16], acc)

                    return carry2

                lax.fori_loop(0, RPC, rowgrp, 0)
                pltpu.async_copy(obuf, acc_sh.at[dlist_v.at[ch]], ssems[par],
                                 add=True)
            return carry

        lax.fori_loop(0, GCH // 2, process2, 0)
        for par in range(2):
            pltpu.make_async_copy(obufs[par],
                                  acc_sh.at[dlist_v.at[GCH - 2 + par]],
                                  ssems[par]).wait()
        plsc.subcore_barrier()
        pltpu.sync_copy(acc_sh.at[pl.ds(row0, TROWS)],
                        out_hbm.at[cid * NF + f, pl.ds(row0, TROWS)])
        plsc.subcore_barrier()


def _run_sc(glist, dlist, w8, table, zrows):
    mesh = plsc.VectorSubcoreMesh(core_axis_name="c", subcore_axis_name="s")
    f = functools.partial(
        pl.kernel,
        out_type=jax.ShapeDtypeStruct((NSC * NF, NN, FC), jnp.float32),
        mesh=mesh,
        compiler_params=pltpu.CompilerParams(needs_layout_passes=False),
        scratch_types=[
            pltpu.VMEM((GCH, GW), jnp.int32),
            pltpu.VMEM((GCH, GW), jnp.int32),
            pltpu.VMEM((TROWS, 8), jnp.float32),
            pltpu.VMEM((GW, FC), jnp.float32),
            pltpu.VMEM((GW, FC), jnp.float32),
            pltpu.VMEM((GW, FC), jnp.float32),
            pltpu.VMEM((GW, FC), jnp.float32),
            pltpu.VMEM_SHARED((NN, FC), jnp.float32),
            pltpu.SemaphoreType.DMA,
            pltpu.SemaphoreType.DMA,
            pltpu.SemaphoreType.DMA,
            pltpu.SemaphoreType.DMA,
        ],
    )(_sc_body)
    return f(glist, dlist, w8, table, zrows)


def _gelu_exact(x):
    return 0.5 * x * (1.0 + lax.erf(x * 0.7071067811865476))


def _tail_body(y6_ref, s_ref, emb_ref, wg_ref, bg_ref, w1_ref, b1_ref,
               w2_ref, b2_ref, wh_ref, bh_ref, out_ref):
    hp = jax.lax.Precision.HIGHEST
    s = jnp.maximum(s_ref[...][:, KTOP:KTOP + 1], 1e-8)
    y6 = y6_ref[...]
    y = jnp.concatenate([y6[q] for q in range(NSC * NF)], axis=1) / s
    emb = emb_ref[...]
    h = _gelu_exact(
        jnp.dot(y, wg_ref[...], preferred_element_type=jnp.float32,
                precision=hp) + bg_ref[...]) + emb
    h1 = _gelu_exact(
        jnp.dot(h, w1_ref[...], preferred_element_type=jnp.float32,
                precision=hp) + b1_ref[...])
    h2 = _gelu_exact(
        jnp.dot(h1, w2_ref[...], preferred_element_type=jnp.float32,
                precision=hp) + b2_ref[...])
    r = jnp.dot(h2, wh_ref[...], preferred_element_type=jnp.float32,
                precision=hp) + bh_ref[...]
    sp = jax.nn.softplus(r)
    col = lax.broadcasted_iota(jnp.int32, r.shape, 1)
    out8 = jnp.where(col == 0, r,
                     jnp.where(col == 2, jnp.minimum(sp, 28.0) + 1.01,
                               sp + 1e-6))
    out_ref[...] = out8[:, :4]


def _run_tail(out6, w8, emb, Wg_t, bg2, W1_t, b12, W2_t, b22, Wh_t, bh2):
    whole = lambda shape: pl.BlockSpec(shape, lambda i: tuple(0 for _ in shape))
    return pl.pallas_call(
        _tail_body,
        grid=(NN // RB,),
        in_specs=[
            pl.BlockSpec((NSC * NF, RB, FC), lambda i: (0, i, 0)),
            pl.BlockSpec((RB, 8), lambda i: (i, 0)),
            pl.BlockSpec((RB, DD), lambda i: (i, 0)),
            whole((DD, DD)), whole((1, DD)),
            whole((DD, 512)), whole((1, 512)),
            whole((512, 256)), whole((1, 256)),
            whole((256, 8)), whole((1, 8)),
        ],
        out_specs=pl.BlockSpec((RB, 4), lambda i: (i, 0)),
        out_shape=jax.ShapeDtypeStruct((NN, 4), jnp.float32),
    )(out6, w8, emb, Wg_t, bg2, W1_t, b12, W2_t, b22, Wh_t, bh2)


def kernel(emb, A, W_gcn, b_gcn, W1, b1, W2, b2, Wh, bh):
    ti8, v8 = _run_topk(A)
    w8 = _run_ws(A, ti8, v8)
    table = _run_table(emb).reshape(NSC * NF * NN, FC)

    rows = jnp.arange(NN, dtype=jnp.int32)[:, None]
    ti5 = ti8[:, :KTOP]
    base6 = jnp.concatenate([rows, ti5], axis=1).reshape(-1)
    dlist = base6.reshape(NTILE, GCH, GW)
    qoff = (jnp.arange(NSC * NF, dtype=jnp.int32) * NN).reshape(
        NSC, NF, 1, 1, 1)
    glist = base6.reshape(1, 1, NTILE, GCH, GW) + qoff
    zrows = jnp.zeros((TROWS, FC), jnp.float32)

    out6 = _run_sc(glist, dlist, w8, table, zrows)

    Wg_t = W_gcn.T
    W1_t = W1.T
    W2_t = W2.T
    Wh_t = jnp.pad(Wh, ((0, 4), (0, 0))).T
    bh2 = jnp.pad(bh, (0, 4)).reshape(1, 8)
    return _run_tail(out6, w8, emb, Wg_t, b_gcn.reshape(1, DD), W1_t,
                     b1.reshape(1, 512), W2_t, b2.reshape(1, 256), Wh_t, bh2)


# speedup vs baseline: 4.4771x; 1.1017x over previous
import functools

import jax
import jax.numpy as jnp
from jax import lax
from jax.experimental import pallas as pl
from jax.experimental.pallas import tpu as pltpu
from jax.experimental.pallas import tpu_sc as plsc

NN = 4096
DD = 768
KTOP = 5
RB = 256

NSC = 2
NTILE = 16
TROWS = NN // NTILE
RPC = 8
GW = RPC * (KTOP + 1)
GCH = TROWS // RPC
FC = 128
NF = 3


def _topk_body(a_ref, ti_ref, v_ref):
    blk = a_ref[...]
    cols = lax.broadcasted_iota(jnp.int32, blk.shape, 1)
    cur = blk
    vs, idxs = [], []
    for _ in range(KTOP):
        m = jnp.max(cur, axis=1, keepdims=True)
        idx = jnp.min(jnp.where(cur == m, cols, blk.shape[1]), axis=1,
                      keepdims=True)
        vs.append(m)
        idxs.append(idx)
        cur = jnp.where(cols == idx, -jnp.inf, cur)
    v5 = jnp.concatenate(vs, axis=1)
    i5 = jnp.concatenate(idxs, axis=1)
    t = v5[:, KTOP - 1:KTOP]
    c = jnp.max(jnp.where(v5 == t, i5, -1), axis=1, keepdims=True)
    ipad = jnp.zeros((blk.shape[0], 2), jnp.int32)
    fpad = jnp.zeros((blk.shape[0], 3), jnp.float32)
    ti_ref[...] = jnp.concatenate([i5, c, ipad], axis=1)
    v_ref[...] = jnp.concatenate([v5, fpad], axis=1)


def _run_topk(A):
    return pl.pallas_call(
        _topk_body,
        grid=(NN // RB,),
        in_specs=[pl.BlockSpec((RB, NN), lambda i: (i, 0))],
        out_specs=[pl.BlockSpec((RB, 8), lambda i: (i, 0)),
                   pl.BlockSpec((RB, 8), lambda i: (i, 0))],
        out_shape=[jax.ShapeDtypeStruct((NN, 8), jnp.int32),
                   jax.ShapeDtypeStruct((NN, 8), jnp.float32)],
    )(A)


def _ws_body(a_ref, ti_ref, v_ref, t2_ref, c2_ref, w_ref):
    i = pl.program_id(0)
    blk = a_ref[...]
    t2 = t2_ref[...]
    c2 = c2_ref[...]
    cols = lax.broadcasted_iota(jnp.int32, blk.shape, 1)
    rows = i * RB + lax.broadcasted_iota(jnp.int32, blk.shape, 0)
    t_i = v_ref[...][:, KTOP - 1:KTOP]
    c_i = ti_ref[...][:, KTOP:KTOP + 1]
    inrow = (blk > t_i) | ((blk == t_i) & (cols <= c_i))
    incol = (blk > t2) | ((blk == t2) & (rows <= c2))
    s = 1.0 + jnp.sum(jnp.where(inrow | incol, blk, 0.0), axis=1,
                      keepdims=True)
    ti8 = ti_ref[...]
    v8 = v_ref[...]
    incf = jnp.where(incol, 1.0, 0.0)
    wcols = []
    for k in range(KTOP):
        tik = ti8[:, k:k + 1]
        mut = jnp.max(jnp.where(cols == tik, incf, 0.0), axis=1,
                      keepdims=True)
        vk = v8[:, k:k + 1]
        wcols.append(vk * (1.0 - 0.5 * mut))
    zp2 = jnp.zeros((blk.shape[0], 2), jnp.float32)
    w_ref[...] = jnp.concatenate(wcols + [s, zp2], axis=1)


def _run_ws(A, ti8, v8):
    t2 = v8[:, KTOP - 1].reshape(1, NN)
    c2 = ti8[:, KTOP].reshape(1, NN)
    return pl.pallas_call(
        _ws_body,
        grid=(NN // RB,),
        in_specs=[pl.BlockSpec((RB, NN), lambda i: (i, 0)),
                  pl.BlockSpec((RB, 8), lambda i: (i, 0)),
                  pl.BlockSpec((RB, 8), lambda i: (i, 0)),
                  pl.BlockSpec((1, NN), lambda i: (0, 0)),
                  pl.BlockSpec((1, NN), lambda i: (0, 0))],
        out_specs=pl.BlockSpec((RB, 8), lambda i: (i, 0)),
        out_shape=jax.ShapeDtypeStruct((NN, 8), jnp.float32),
    )(A, ti8, v8, t2, c2)


def _table_body(e_ref, t_ref):
    e = e_ref[...]
    t_ref[...] = e.reshape(RB, NSC * NF, FC).transpose(1, 0, 2)


def _run_table(emb):
    return pl.pallas_call(
        _table_body,
        grid=(NN // RB,),
        in_specs=[pl.BlockSpec((RB, DD), lambda i: (i, 0))],
        out_specs=pl.BlockSpec((NSC * NF, RB, FC), lambda i: (0, i, 0)),
        out_shape=jax.ShapeDtypeStruct((NSC * NF, NN, FC), jnp.float32),
    )(emb)


def _sc_body(glist_hbm, dlist_hbm, w_hbm, table_hbm, z_hbm, out_hbm,
             glist_v, dlist_v, w_v,
             gbuf0, gbuf1, obuf0, obuf1, acc_sh,
             gsem0, gsem1, ssem0, ssem1):
    cid = lax.axis_index("c")
    sid = lax.axis_index("s")
    row0 = sid * TROWS
    lanes = lax.iota(jnp.int32, 16)
    gbufs = (gbuf0, gbuf1)
    obufs = (obuf0, obuf1)
    gsems = (gsem0, gsem1)
    ssems = (ssem0, ssem1)

    pltpu.sync_copy(dlist_hbm.at[sid], dlist_v)
    pltpu.sync_copy(w_hbm.at[pl.ds(row0, TROWS)], w_v)

    for f in range(NF):
        pltpu.sync_copy(glist_hbm.at[cid, f, sid], glist_v)
        pltpu.sync_copy(z_hbm, acc_sh.at[pl.ds(row0, TROWS)])
        plsc.subcore_barrier()

        pltpu.async_copy(table_hbm.at[glist_v.at[0]], gbuf0, gsem0)

        def process2(i2, carry):
            for par in range(2):
                ch = i2 * 2 + par
                gbuf = gbufs[par]
                obuf = obufs[par]
                pltpu.make_async_copy(table_hbm.at[glist_v.at[ch]], gbuf,
                                      gsems[par]).wait()

                @pl.when(ch + 1 < GCH)
                def _():
                    pltpu.async_copy(table_hbm.at[glist_v.at[ch + 1]],
                                     gbufs[1 - par], gsems[1 - par])

                @pl.when(ch >= 2)
                def _():
                    pltpu.make_async_copy(obuf, acc_sh.at[dlist_v.at[ch - 2]],
                                          ssems[par]).wait()

                def rowgrp(r, carry2):
                    b = r * (KTOP + 1)
                    rloc = ch * RPC + r
                    rsp = jnp.full((16,), rloc, jnp.int32)
                    ws = [plsc.load_gather(
                        w_v, [rsp, jnp.full((16,), k, jnp.int32)])
                        for k in range(KTOP)]

                    @functools.partial(plsc.parallel_loop, 0, FC // 16,
                                       unroll=4)
                    def colq(q):
                        c16 = q * 16 + lanes
                        b16 = jnp.full((16,), b, jnp.int32)
                        g0 = plsc.load_gather(gbuf, [b16, c16])
                        acc = g0
                        for k in range(KTOP):
                            bk = jnp.full((16,), b + 1 + k, jnp.int32)
                            gk = plsc.load_gather(gbuf, [bk, c16])
                            acc = acc + ws[k] * gk
                            plsc.store_scatter(obuf, [bk, c16], ws[k] * g0)
                        plsc.store_scatter(obuf, [b16, c16], acc)

                    return carry2

                lax.fori_loop(0, RPC, rowgrp, 0)
                pltpu.async_copy(obuf, acc_sh.at[dlist_v.at[ch]], ssems[par],
                                 add=True)
            return carry

        lax.fori_loop(0, GCH // 2, process2, 0)
        for par in range(2):
            pltpu.make_async_copy(obufs[par],
                                  acc_sh.at[dlist_v.at[GCH - 2 + par]],
                                  ssems[par]).wait()
        plsc.subcore_barrier()
        pltpu.sync_copy(acc_sh.at[pl.ds(row0, TROWS)],
                        out_hbm.at[cid * NF + f, pl.ds(row0, TROWS)])
        plsc.subcore_barrier()


def _run_sc(glist, dlist, w8, table, zrows):
    mesh = plsc.VectorSubcoreMesh(core_axis_name="c", subcore_axis_name="s")
    f = functools.partial(
        pl.kernel,
        out_type=jax.ShapeDtypeStruct((NSC * NF, NN, FC), jnp.float32),
        mesh=mesh,
        compiler_params=pltpu.CompilerParams(needs_layout_passes=False),
        scratch_types=[
            pltpu.VMEM((GCH, GW), jnp.int32),
            pltpu.VMEM((GCH, GW), jnp.int32),
            pltpu.VMEM((TROWS, 8), jnp.float32),
            pltpu.VMEM((GW, FC), jnp.float32),
            pltpu.VMEM((GW, FC), jnp.float32),
            pltpu.VMEM((GW, FC), jnp.float32),
            pltpu.VMEM((GW, FC), jnp.float32),
            pltpu.VMEM_SHARED((NN, FC), jnp.float32),
            pltpu.SemaphoreType.DMA,
            pltpu.SemaphoreType.DMA,
            pltpu.SemaphoreType.DMA,
            pltpu.SemaphoreType.DMA,
        ],
    )(_sc_body)
    return f(glist, dlist, w8, table, zrows)


def _gelu_exact(x):
    return 0.5 * x * (1.0 + lax.erf(x * 0.7071067811865476))


def _tail_body(y6_ref, s_ref, emb_ref, wg_ref, bg_ref, w1_ref, b1_ref,
               w2_ref, b2_ref, wh_ref, bh_ref, out_ref):
    def matT(x, w_ref2):
        return lax.dot_general(x, w_ref2[...], (((1,), (1,)), ((), ())),
                               preferred_element_type=jnp.float32,
                               precision=jax.lax.Precision.HIGHEST)
    s = jnp.maximum(s_ref[...][:, KTOP:KTOP + 1], 1e-8)
    y6 = y6_ref[...]
    y = jnp.concatenate([y6[q] for q in range(NSC * NF)], axis=1) / s
    emb = emb_ref[...]
    h = _gelu_exact(matT(y, wg_ref) + bg_ref[...]) + emb
    h1 = _gelu_exact(matT(h, w1_ref) + b1_ref[...])
    h2 = _gelu_exact(matT(h1, w2_ref) + b2_ref[...])
    r = matT(h2, wh_ref) + bh_ref[...]
    sp = jax.nn.softplus(r)
    col = lax.broadcasted_iota(jnp.int32, r.shape, 1)
    out8 = jnp.where(col == 0, r,
                     jnp.where(col == 2, jnp.minimum(sp, 28.0) + 1.01,
                               sp + 1e-6))
    out_ref[...] = out8[:, :4]


def _run_tail(out6, w8, emb, Wg, bg2, W1, b12, W2, b22, Whp, bh2):
    whole = lambda shape: pl.BlockSpec(shape, lambda i: tuple(0 for _ in shape))
    return pl.pallas_call(
        _tail_body,
        grid=(NN // RB,),
        in_specs=[
            pl.BlockSpec((NSC * NF, RB, FC), lambda i: (0, i, 0)),
            pl.BlockSpec((RB, 8), lambda i: (i, 0)),
            pl.BlockSpec((RB, DD), lambda i: (i, 0)),
            whole((DD, DD)), whole((1, DD)),
            whole((512, DD)), whole((1, 512)),
            whole((256, 512)), whole((1, 256)),
            whole((8, 256)), whole((1, 8)),
        ],
        out_specs=pl.BlockSpec((RB, 4), lambda i: (i, 0)),
        out_shape=jax.ShapeDtypeStruct((NN, 4), jnp.float32),
    )(out6, w8, emb, Wg, bg2, W1, b12, W2, b22, Whp, bh2)


def kernel(emb, A, W_gcn, b_gcn, W1, b1, W2, b2, Wh, bh):
    ti8, v8 = _run_topk(A)
    w8 = _run_ws(A, ti8, v8)
    table = _run_table(emb).reshape(NSC * NF * NN, FC)

    rows = jnp.arange(NN, dtype=jnp.int32)[:, None]
    ti5 = ti8[:, :KTOP]
    base6 = jnp.concatenate([rows, ti5], axis=1).reshape(-1)
    dlist = base6.reshape(NTILE, GCH, GW)
    qoff = (jnp.arange(NSC * NF, dtype=jnp.int32) * NN).reshape(
        NSC, NF, 1, 1, 1)
    glist = base6.reshape(1, 1, NTILE, GCH, GW) + qoff
    zrows = jnp.zeros((TROWS, FC), jnp.float32)

    out6 = _run_sc(glist, dlist, w8, table, zrows)

    Whp = jnp.pad(Wh, ((0, 4), (0, 0)))
    bh2 = jnp.pad(bh, (0, 4)).reshape(1, 8)
    return _run_tail(out6, w8, emb, W_gcn, b_gcn.reshape(1, DD), W1,
                     b1.reshape(1, 512), W2, b2.reshape(1, 256), Whp, bh2)


# speedup vs baseline: 5.1266x; 1.1451x over previous
import functools

import jax
import jax.numpy as jnp
from jax import lax
from jax.experimental import pallas as pl
from jax.experimental.pallas import tpu as pltpu
from jax.experimental.pallas import tpu_sc as plsc

NN = 4096
DD = 768
KTOP = 5
RB = 256

NSC = 2
NTILE = 16
TROWS = NN // NTILE
RPC = 8
GW = RPC * (KTOP + 1)
GCH = TROWS // RPC
FC = 128
NF = 3


def _topk_body(a_ref, ti_ref, v_ref, cp_ref):
    blk = a_ref[...]
    cols = lax.broadcasted_iota(jnp.int32, blk.shape, 1)
    cur = blk
    vs, idxs = [], []
    for _ in range(KTOP):
        m = jnp.max(cur, axis=1, keepdims=True)
        idx = jnp.min(jnp.where(cur == m, cols, blk.shape[1]), axis=1,
                      keepdims=True)
        vs.append(m)
        idxs.append(idx)
        cur = jnp.where(cols == idx, -jnp.inf, cur)
    v5 = jnp.concatenate(vs, axis=1)
    i5 = jnp.concatenate(idxs, axis=1)
    t = v5[:, KTOP - 1:KTOP]
    c = jnp.max(jnp.where(v5 == t, i5, -1), axis=1, keepdims=True)
    ipad = jnp.zeros((blk.shape[0], 2), jnp.int32)
    fpad = jnp.zeros((blk.shape[0], 3), jnp.float32)
    ti_ref[...] = jnp.concatenate([i5, c, ipad], axis=1)
    v_ref[...] = jnp.concatenate([v5, fpad], axis=1)
    inrow = (blk > t) | ((blk == t) & (cols <= c))
    cp_ref[...] = jnp.sum(jnp.where(inrow, blk, 0.0), axis=0,
                          keepdims=True).reshape(1, 1, blk.shape[1])


def _run_topk(A):
    return pl.pallas_call(
        _topk_body,
        grid=(NN // RB,),
        in_specs=[pl.BlockSpec((RB, NN), lambda i: (i, 0))],
        out_specs=[pl.BlockSpec((RB, 8), lambda i: (i, 0)),
                   pl.BlockSpec((RB, 8), lambda i: (i, 0)),
                   pl.BlockSpec((1, 1, NN), lambda i: (i, 0, 0))],
        out_shape=[jax.ShapeDtypeStruct((NN, 8), jnp.int32),
                   jax.ShapeDtypeStruct((NN, 8), jnp.float32),
                   jax.ShapeDtypeStruct((NN // RB, 1, NN), jnp.float32)],
    )(A)


TW = NN // (NSC * NTILE)


def _scw_body(ti_hbm, v_hbm, t_hbm, c_hbm, w_hbm,
              ti_v, v_v, t_v, c_v, w_v):
    cid = lax.axis_index("c")
    sid = lax.axis_index("s")
    wid = sid * NSC + cid
    row0 = wid * TW
    lanes = lax.iota(jnp.int32, 16)
    pltpu.sync_copy(ti_hbm.at[pl.ds(row0, TW)], ti_v)
    pltpu.sync_copy(v_hbm.at[pl.ds(row0, TW)], v_v)
    pltpu.sync_copy(t_hbm, t_v)
    pltpu.sync_copy(c_hbm, c_v)

    def build_w(ec, carry):
        e16 = ec * 16 + lanes
        r16 = e16 // KTOP
        k16 = e16 - r16 * KTOP
        j16 = plsc.load_gather(ti_v, [r16, k16])
        v16 = plsc.load_gather(v_v, [r16, k16])
        tj = plsc.load_gather(t_v, [j16])
        cj = plsc.load_gather(c_v, [j16])
        i16 = row0 + r16
        mut = (v16 > tj) | ((v16 == tj) & (i16 <= cj))
        w16 = v16 * jnp.where(mut, 0.5, 1.0)
        plsc.store_scatter(w_v, [r16, k16], w16)
        return carry

    lax.fori_loop(0, TW * KTOP // 16, build_w, 0)

    def zpad(ec, carry):
        r16 = ec * 16 + lanes
        z = jnp.zeros((16,), jnp.float32)
        for k in range(KTOP, 8):
            plsc.store_scatter(w_v, [r16, jnp.full((16,), k, jnp.int32)], z)
        return carry

    lax.fori_loop(0, TW // 16, zpad, 0)
    pltpu.sync_copy(w_v, w_hbm.at[pl.ds(row0, TW)])


def _run_scw(ti8, v8, tvec, cvec):
    mesh = plsc.VectorSubcoreMesh(core_axis_name="c", subcore_axis_name="s")
    f = functools.partial(
        pl.kernel,
        out_type=jax.ShapeDtypeStruct((NN, 8), jnp.float32),
        mesh=mesh,
        compiler_params=pltpu.CompilerParams(needs_layout_passes=False),
        scratch_types=[
            pltpu.VMEM((TW, 8), jnp.int32),
            pltpu.VMEM((TW, 8), jnp.float32),
            pltpu.VMEM((NN,), jnp.float32),
            pltpu.VMEM((NN,), jnp.int32),
            pltpu.VMEM((TW, 8), jnp.float32),
        ],
    )(_scw_body)
    return f(ti8, v8, tvec, cvec)


def _table_body(e_ref, t_ref):
    e = e_ref[...]
    t_ref[...] = e.reshape(RB, NSC * NF, FC).transpose(1, 0, 2)


def _run_table(emb):
    return pl.pallas_call(
        _table_body,
        grid=(NN // RB,),
        in_specs=[pl.BlockSpec((RB, DD), lambda i: (i, 0))],
        out_specs=pl.BlockSpec((NSC * NF, RB, FC), lambda i: (0, i, 0)),
        out_shape=jax.ShapeDtypeStruct((NSC * NF, NN, FC), jnp.float32),
    )(emb)


def _sc_body(glist_hbm, dlist_hbm, w_hbm, table_hbm, z_hbm, out_hbm,
             glist_v, dlist_v, w_v,
             gbuf0, gbuf1, obuf0, obuf1, acc_sh,
             gsem0, gsem1, ssem0, ssem1):
    cid = lax.axis_index("c")
    sid = lax.axis_index("s")
    row0 = sid * TROWS
    lanes = lax.iota(jnp.int32, 16)
    gbufs = (gbuf0, gbuf1)
    obufs = (obuf0, obuf1)
    gsems = (gsem0, gsem1)
    ssems = (ssem0, ssem1)

    pltpu.sync_copy(dlist_hbm.at[sid], dlist_v)
    pltpu.sync_copy(w_hbm.at[pl.ds(row0, TROWS)], w_v)

    for f in range(NF):
        pltpu.sync_copy(glist_hbm.at[cid, f, sid], glist_v)
        pltpu.sync_copy(z_hbm, acc_sh.at[pl.ds(row0, TROWS)])
        plsc.subcore_barrier()

        pltpu.async_copy(table_hbm.at[glist_v.at[0]], gbuf0, gsem0)

        def process2(i2, carry):
            for par in range(2):
                ch = i2 * 2 + par
                gbuf = gbufs[par]
                obuf = obufs[par]
                pltpu.make_async_copy(table_hbm.at[glist_v.at[ch]], gbuf,
                                      gsems[par]).wait()

                @pl.when(ch + 1 < GCH)
                def _():
                    pltpu.async_copy(table_hbm.at[glist_v.at[ch + 1]],
                                     gbufs[1 - par], gsems[1 - par])

                @pl.when(ch >= 2)
                def _():
                    pltpu.make_async_copy(obuf, acc_sh.at[dlist_v.at[ch - 2]],
                                          ssems[par]).wait()

                def rowgrp(r, carry2):
                    b = r * (KTOP + 1)
                    rloc = ch * RPC + r
                    rsp = jnp.full((16,), rloc, jnp.int32)
                    ws = [plsc.load_gather(
                        w_v, [rsp, jnp.full((16,), k, jnp.int32)])
                        for k in range(KTOP)]

                    @functools.partial(plsc.parallel_loop, 0, FC // 16,
                                       unroll=4)
                    def colq(q):
                        c16 = q * 16 + lanes
                        b16 = jnp.full((16,), b, jnp.int32)
                        g0 = plsc.load_gather(gbuf, [b16, c16])
                        acc = g0
                        for k in range(KTOP):
                            bk = jnp.full((16,), b + 1 + k, jnp.int32)
                            gk = plsc.load_gather(gbuf, [bk, c16])
                            acc = acc + ws[k] * gk
                            plsc.store_scatter(obuf, [bk, c16], ws[k] * g0)
                        plsc.store_scatter(obuf, [b16, c16], acc)

                    return carry2

                lax.fori_loop(0, RPC, rowgrp, 0)
                pltpu.async_copy(obuf, acc_sh.at[dlist_v.at[ch]], ssems[par],
                                 add=True)
            return carry

        lax.fori_loop(0, GCH // 2, process2, 0)
        for par in range(2):
            pltpu.make_async_copy(obufs[par],
                                  acc_sh.at[dlist_v.at[GCH - 2 + par]],
                                  ssems[par]).wait()
        plsc.subcore_barrier()
        pltpu.sync_copy(acc_sh.at[pl.ds(row0, TROWS)],
                        out_hbm.at[cid * NF + f, pl.ds(row0, TROWS)])
        plsc.subcore_barrier()


def _run_sc(glist, dlist, w8, table, zrows):
    mesh = plsc.VectorSubcoreMesh(core_axis_name="c", subcore_axis_name="s")
    f = functools.partial(
        pl.kernel,
        out_type=jax.ShapeDtypeStruct((NSC * NF, NN, FC), jnp.float32),
        mesh=mesh,
        compiler_params=pltpu.CompilerParams(needs_layout_passes=False),
        scratch_types=[
            pltpu.VMEM((GCH, GW), jnp.int32),
            pltpu.VMEM((GCH, GW), jnp.int32),
            pltpu.VMEM((TROWS, 8), jnp.float32),
            pltpu.VMEM((GW, FC), jnp.float32),
            pltpu.VMEM((GW, FC), jnp.float32),
            pltpu.VMEM((GW, FC), jnp.float32),
            pltpu.VMEM((GW, FC), jnp.float32),
            pltpu.VMEM_SHARED((NN, FC), jnp.float32),
            pltpu.SemaphoreType.DMA,
            pltpu.SemaphoreType.DMA,
            pltpu.SemaphoreType.DMA,
            pltpu.SemaphoreType.DMA,
        ],
    )(_sc_body)
    return f(glist, dlist, w8, table, zrows)


def _gelu_exact(x):
    return 0.5 * x * (1.0 + lax.erf(x * 0.7071067811865476))


def _tail_body(y6_ref, w_ref, v_ref, cp_ref, emb_ref, wg_ref, bg_ref,
               w1_ref, b1_ref, w2_ref, b2_ref, wh_ref, bh_ref, out_ref):
    def matT(x, w_ref2):
        return lax.dot_general(x, w_ref2[...], (((1,), (1,)), ((), ())),
                               preferred_element_type=jnp.float32,
                               precision=jax.lax.Precision.HIGHEST)
    w8 = w_ref[...]
    v8 = v_ref[...]
    sumw = jnp.sum(w8[:, :KTOP], axis=1, keepdims=True)
    sumv = jnp.sum(v8[:, :KTOP], axis=1, keepdims=True)
    revs = jnp.sum(cp_ref[...], axis=1, keepdims=True)
    s = jnp.maximum(1.0 + revs + 2.0 * sumw - sumv, 1e-8)
    y6 = y6_ref[...]
    y = jnp.concatenate([y6[q] for q in range(NSC * NF)], axis=1) / s
    emb = emb_ref[...]
    h = _gelu_exact(matT(y, wg_ref) + bg_ref[...]) + emb
    h1 = _gelu_exact(matT(h, w1_ref) + b1_ref[...])
    h2 = _gelu_exact(matT(h1, w2_ref) + b2_ref[...])
    r = matT(h2, wh_ref) + bh_ref[...]
    sp = jax.nn.softplus(r)
    col = lax.broadcasted_iota(jnp.int32, r.shape, 1)
    out8 = jnp.where(col == 0, r,
                     jnp.where(col == 2, jnp.minimum(sp, 28.0) + 1.01,
                               sp + 1e-6))
    out_ref[...] = out8[:, :4]


def _run_tail(out6, w8, v8, cpt, emb, Wg, bg2, W1, b12, W2, b22, Whp, bh2):
    whole = lambda shape: pl.BlockSpec(shape, lambda i: tuple(0 for _ in shape))
    return pl.pallas_call(
        _tail_body,
        grid=(NN // RB,),
        in_specs=[
            pl.BlockSpec((NSC * NF, RB, FC), lambda i: (0, i, 0)),
            pl.BlockSpec((RB, 8), lambda i: (i, 0)),
            pl.BlockSpec((RB, 8), lambda i: (i, 0)),
            pl.BlockSpec((RB, NN // RB), lambda i: (i, 0)),
            pl.BlockSpec((RB, DD), lambda i: (i, 0)),
            whole((DD, DD)), whole((1, DD)),
            whole((512, DD)), whole((1, 512)),
            whole((256, 512)), whole((1, 256)),
            whole((8, 256)), whole((1, 8)),
        ],
        out_specs=pl.BlockSpec((RB, 4), lambda i: (i, 0)),
        out_shape=jax.ShapeDtypeStruct((NN, 4), jnp.float32),
    )(out6, w8, v8, cpt, emb, Wg, bg2, W1, b12, W2, b22, Whp, bh2)


def kernel(emb, A, W_gcn, b_gcn, W1, b1, W2, b2, Wh, bh):
    ti8, v8, colpart = _run_topk(A)
    tvec = v8[:, KTOP - 1]
    cvec = ti8[:, KTOP]
    w8 = _run_scw(ti8, v8, tvec, cvec)
    cpt = colpart.reshape(NN // RB, NN).T
    table = _run_table(emb).reshape(NSC * NF * NN, FC)

    rows = jnp.arange(NN, dtype=jnp.int32)[:, None]
    ti5 = ti8[:, :KTOP]
    base6 = jnp.concatenate([rows, ti5], axis=1).reshape(-1)
    dlist = base6.reshape(NTILE, GCH, GW)
    qoff = (jnp.arange(NSC * NF, dtype=jnp.int32) * NN).reshape(
        NSC, NF, 1, 1, 1)
    glist = base6.reshape(1, 1, NTILE, GCH, GW) + qoff
    zrows = jnp.zeros((TROWS, FC), jnp.float32)

    out6 = _run_sc(glist, dlist, w8, table, zrows)

    Whp = jnp.pad(Wh, ((0, 4), (0, 0)))
    bh2 = jnp.pad(bh, (0, 4)).reshape(1, 8)
    return _run_tail(out6, w8, v8, cpt, emb, W_gcn, b_gcn.reshape(1, DD),
                     W1, b1.reshape(1, 512), W2, b2.reshape(1, 256), Whp,
                     bh2)


# speedup vs baseline: 5.2547x; 1.0250x over previous
import functools

import jax
import jax.numpy as jnp
from jax import lax
from jax.experimental import pallas as pl
from jax.experimental.pallas import tpu as pltpu
from jax.experimental.pallas import tpu_sc as plsc

NN = 4096
DD = 768
KTOP = 5
RB = 256

NSC = 2
NTILE = 16
TROWS = NN // NTILE
RPC = 8
GW = RPC * (KTOP + 1)
GCH = TROWS // RPC
FC = 128
NF = 3


def _topk_body(a_ref, e_ref, ti_ref, v_ref, cp_ref, tbl_ref):
    blk = a_ref[...]
    cols = lax.broadcasted_iota(jnp.int32, blk.shape, 1)
    cur = blk
    vs, idxs = [], []
    for _ in range(KTOP):
        m = jnp.max(cur, axis=1, keepdims=True)
        idx = jnp.min(jnp.where(cur == m, cols, blk.shape[1]), axis=1,
                      keepdims=True)
        vs.append(m)
        idxs.append(idx)
        cur = jnp.where(cols == idx, -jnp.inf, cur)
    v5 = jnp.concatenate(vs, axis=1)
    i5 = jnp.concatenate(idxs, axis=1)
    t = v5[:, KTOP - 1:KTOP]
    c = jnp.max(jnp.where(v5 == t, i5, -1), axis=1, keepdims=True)
    ipad = jnp.zeros((blk.shape[0], 2), jnp.int32)
    fpad = jnp.zeros((blk.shape[0], 3), jnp.float32)
    ti_ref[...] = jnp.concatenate([i5, c, ipad], axis=1)
    v_ref[...] = jnp.concatenate([v5, fpad], axis=1)
    inrow = (blk > t) | ((blk == t) & (cols <= c))
    cp_ref[...] = jnp.sum(jnp.where(inrow, blk, 0.0), axis=0,
                          keepdims=True).reshape(1, 1, blk.shape[1])
    e = e_ref[...]
    tbl_ref[...] = e.reshape(e.shape[0], NSC * NF, FC).transpose(1, 0, 2)


def _run_topk(A, emb):
    return pl.pallas_call(
        _topk_body,
        grid=(NN // RB,),
        in_specs=[pl.BlockSpec((RB, NN), lambda i: (i, 0)),
                  pl.BlockSpec((RB, DD), lambda i: (i, 0))],
        out_specs=[pl.BlockSpec((RB, 8), lambda i: (i, 0)),
                   pl.BlockSpec((RB, 8), lambda i: (i, 0)),
                   pl.BlockSpec((1, 1, NN), lambda i: (i, 0, 0)),
                   pl.BlockSpec((NSC * NF, RB, FC), lambda i: (0, i, 0))],
        out_shape=[jax.ShapeDtypeStruct((NN, 8), jnp.int32),
                   jax.ShapeDtypeStruct((NN, 8), jnp.float32),
                   jax.ShapeDtypeStruct((NN // RB, 1, NN), jnp.float32),
                   jax.ShapeDtypeStruct((NSC * NF, NN, FC), jnp.float32)],
    )(A, emb)


TW = NN // (NSC * NTILE)


def _scw_body(ti_hbm, v_hbm, t_hbm, c_hbm, w_hbm,
              ti_v, v_v, t_v, c_v, w_v):
    cid = lax.axis_index("c")
    sid = lax.axis_index("s")
    wid = sid * NSC + cid
    row0 = wid * TW
    lanes = lax.iota(jnp.int32, 16)
    pltpu.sync_copy(ti_hbm.at[pl.ds(row0, TW)], ti_v)
    pltpu.sync_copy(v_hbm.at[pl.ds(row0, TW)], v_v)
    pltpu.sync_copy(t_hbm, t_v)
    pltpu.sync_copy(c_hbm, c_v)

    def build_w(ec, carry):
        e16 = ec * 16 + lanes
        r16 = e16 // KTOP
        k16 = e16 - r16 * KTOP
        j16 = plsc.load_gather(ti_v, [r16, k16])
        v16 = plsc.load_gather(v_v, [r16, k16])
        tj = plsc.load_gather(t_v, [j16])
        cj = plsc.load_gather(c_v, [j16])
        i16 = row0 + r16
        mut = (v16 > tj) | ((v16 == tj) & (i16 <= cj))
        w16 = v16 * jnp.where(mut, 0.5, 1.0)
        plsc.store_scatter(w_v, [r16, k16], w16)
        return carry

    lax.fori_loop(0, TW * KTOP // 16, build_w, 0)

    def zpad(ec, carry):
        r16 = ec * 16 + lanes
        z = jnp.zeros((16,), jnp.float32)
        for k in range(KTOP, 8):
            plsc.store_scatter(w_v, [r16, jnp.full((16,), k, jnp.int32)], z)
        return carry

    lax.fori_loop(0, TW // 16, zpad, 0)
    pltpu.sync_copy(w_v, w_hbm.at[pl.ds(row0, TW)])


def _run_scw(ti8, v8, tvec, cvec):
    mesh = plsc.VectorSubcoreMesh(core_axis_name="c", subcore_axis_name="s")
    f = functools.partial(
        pl.kernel,
        out_type=jax.ShapeDtypeStruct((NN, 8), jnp.float32),
        mesh=mesh,
        compiler_params=pltpu.CompilerParams(needs_layout_passes=False),
        scratch_types=[
            pltpu.VMEM((TW, 8), jnp.int32),
            pltpu.VMEM((TW, 8), jnp.float32),
            pltpu.VMEM((NN,), jnp.float32),
            pltpu.VMEM((NN,), jnp.int32),
            pltpu.VMEM((TW, 8), jnp.float32),
        ],
    )(_scw_body)
    return f(ti8, v8, tvec, cvec)


TW = NN // (NSC * NTILE)


def _scw_body(ti_hbm, v_hbm, t_hbm, c_hbm, w_hbm,
              ti_v, v_v, t_v, c_v, w_v):
    cid = lax.axis_index("c")
    sid = lax.axis_index("s")
    wid = sid * NSC + cid
    row0 = wid * TW
    lanes = lax.iota(jnp.int32, 16)
    pltpu.sync_copy(ti_hbm.at[pl.ds(row0, TW)], ti_v)
    pltpu.sync_copy(v_hbm.at[pl.ds(row0, TW)], v_v)
    pltpu.sync_copy(t_hbm, t_v)
    pltpu.sync_copy(c_hbm, c_v)

    def build_w(ec, carry):
        e16 = ec * 16 + lanes
        r16 = e16 // KTOP
        k16 = e16 - r16 * KTOP
        j16 = plsc.load_gather(ti_v, [r16, k16])
        v16 = plsc.load_gather(v_v, [r16, k16])
        tj = plsc.load_gather(t_v, [j16])
        cj = plsc.load_gather(c_v, [j16])
        i16 = row0 + r16
        mut = (v16 > tj) | ((v16 == tj) & (i16 <= cj))
        w16 = v16 * jnp.where(mut, 0.5, 1.0)
        plsc.store_scatter(w_v, [r16, k16], w16)
        return carry

    lax.fori_loop(0, TW * KTOP // 16, build_w, 0)

    def zpad(ec, carry):
        r16 = ec * 16 + lanes
        z = jnp.zeros((16,), jnp.float32)
        for k in range(KTOP, 8):
            plsc.store_scatter(w_v, [r16, jnp.full((16,), k, jnp.int32)], z)
        return carry

    lax.fori_loop(0, TW // 16, zpad, 0)
    pltpu.sync_copy(w_v, w_hbm.at[pl.ds(row0, TW)])


def _run_scw(ti8, v8, tvec, cvec):
    mesh = plsc.VectorSubcoreMesh(core_axis_name="c", subcore_axis_name="s")
    f = functools.partial(
        pl.kernel,
        out_type=jax.ShapeDtypeStruct((NN, 8), jnp.float32),
        mesh=mesh,
        compiler_params=pltpu.CompilerParams(needs_layout_passes=False),
        scratch_types=[
            pltpu.VMEM((TW, 8), jnp.int32),
            pltpu.VMEM((TW, 8), jnp.float32),
            pltpu.VMEM((NN,), jnp.float32),
            pltpu.VMEM((NN,), jnp.int32),
            pltpu.VMEM((TW, 8), jnp.float32),
        ],
    )(_scw_body)
    return f(ti8, v8, tvec, cvec)


def _table_body(e_ref, t_ref):
    e = e_ref[...]
    t_ref[...] = e.reshape(RB, NSC * NF, FC).transpose(1, 0, 2)


def _run_table(emb):
    return pl.pallas_call(
        _table_body,
        grid=(NN // RB,),
        in_specs=[pl.BlockSpec((RB, DD), lambda i: (i, 0))],
        out_specs=pl.BlockSpec((NSC * NF, RB, FC), lambda i: (0, i, 0)),
        out_shape=jax.ShapeDtypeStruct((NSC * NF, NN, FC), jnp.float32),
    )(emb)


def _sc_body(glist_hbm, dlist_hbm, w_hbm, table_hbm, z_hbm, out_hbm,
             glist_v, dlist_v, w_v,
             gbuf0, gbuf1, obuf0, obuf1, acc_sh,
             gsem0, gsem1, ssem0, ssem1):
    cid = lax.axis_index("c")
    sid = lax.axis_index("s")
    row0 = sid * TROWS
    lanes = lax.iota(jnp.int32, 16)
    gbufs = (gbuf0, gbuf1)
    obufs = (obuf0, obuf1)
    gsems = (gsem0, gsem1)
    ssems = (ssem0, ssem1)

    pltpu.sync_copy(dlist_hbm.at[sid], dlist_v)
    pltpu.sync_copy(w_hbm.at[pl.ds(row0, TROWS)], w_v)

    for f in range(NF):
        pltpu.sync_copy(glist_hbm.at[cid, f, sid], glist_v)
        pltpu.sync_copy(z_hbm, acc_sh.at[pl.ds(row0, TROWS)])
        plsc.subcore_barrier()

        pltpu.async_copy(table_hbm.at[glist_v.at[0]], gbuf0, gsem0)

        def process2(i2, carry):
            for par in range(2):
                ch = i2 * 2 + par
                gbuf = gbufs[par]
                obuf = obufs[par]
                pltpu.make_async_copy(table_hbm.at[glist_v.at[ch]], gbuf,
                                      gsems[par]).wait()

                @pl.when(ch + 1 < GCH)
                def _():
                    pltpu.async_copy(table_hbm.at[glist_v.at[ch + 1]],
                                     gbufs[1 - par], gsems[1 - par])

                @pl.when(ch >= 2)
                def _():
                    pltpu.make_async_copy(obuf, acc_sh.at[dlist_v.at[ch - 2]],
                                          ssems[par]).wait()

                def rowgrp(r, carry2):
                    b = r * (KTOP + 1)
                    rloc = ch * RPC + r
                    rsp = jnp.full((16,), rloc, jnp.int32)
                    ws = [plsc.load_gather(
                        w_v, [rsp, jnp.full((16,), k, jnp.int32)])
                        for k in range(KTOP)]

                    @functools.partial(plsc.parallel_loop, 0, FC // 16,
                                       unroll=4)
                    def colq(q):
                        c16 = q * 16 + lanes
                        b16 = jnp.full((16,), b, jnp.int32)
                        g0 = plsc.load_gather(gbuf, [b16, c16])
                        acc = g0
                        for k in range(KTOP):
                            bk = jnp.full((16,), b + 1 + k, jnp.int32)
                            gk = plsc.load_gather(gbuf, [bk, c16])
                            acc = acc + ws[k] * gk
                            plsc.store_scatter(obuf, [bk, c16], ws[k] * g0)
                        plsc.store_scatter(obuf, [b16, c16], acc)

                    return carry2

                lax.fori_loop(0, RPC, rowgrp, 0)
                pltpu.async_copy(obuf, acc_sh.at[dlist_v.at[ch]], ssems[par],
                                 add=True)
            return carry

        lax.fori_loop(0, GCH // 2, process2, 0)
        for par in range(2):
            pltpu.make_async_copy(obufs[par],
                                  acc_sh.at[dlist_v.at[GCH - 2 + par]],
                                  ssems[par]).wait()
        plsc.subcore_barrier()
        pltpu.sync_copy(acc_sh.at[pl.ds(row0, TROWS)],
                        out_hbm.at[cid * NF + f, pl.ds(row0, TROWS)])
        plsc.subcore_barrier()


def _run_sc(glist, dlist, w8, table, zrows):
    mesh = plsc.VectorSubcoreMesh(core_axis_name="c", subcore_axis_name="s")
    f = functools.partial(
        pl.kernel,
        out_type=jax.ShapeDtypeStruct((NSC * NF, NN, FC), jnp.float32),
        mesh=mesh,
        compiler_params=pltpu.CompilerParams(needs_layout_passes=False),
        scratch_types=[
            pltpu.VMEM((GCH, GW), jnp.int32),
            pltpu.VMEM((GCH, GW), jnp.int32),
            pltpu.VMEM((TROWS, 8), jnp.float32),
            pltpu.VMEM((GW, FC), jnp.float32),
            pltpu.VMEM((GW, FC), jnp.float32),
            pltpu.VMEM((GW, FC), jnp.float32),
            pltpu.VMEM((GW, FC), jnp.float32),
            pltpu.VMEM_SHARED((NN, FC), jnp.float32),
            pltpu.SemaphoreType.DMA,
            pltpu.SemaphoreType.DMA,
            pltpu.SemaphoreType.DMA,
            pltpu.SemaphoreType.DMA,
        ],
    )(_sc_body)
    return f(glist, dlist, w8, table, zrows)


def _gelu_exact(x):
    return 0.5 * x * (1.0 + lax.erf(x * 0.7071067811865476))


def _tail_body(y6_ref, w_ref, v_ref, cp_ref, emb_ref, wg_ref, bg_ref,
               w1_ref, b1_ref, w2_ref, b2_ref, wh_ref, bh_ref, out_ref):
    def matT(x, w_ref2):
        return lax.dot_general(x, w_ref2[...], (((1,), (1,)), ((), ())),
                               preferred_element_type=jnp.float32,
                               precision=jax.lax.Precision.HIGHEST)
    w8 = w_ref[...]
    v8 = v_ref[...]
    sumw = jnp.sum(w8[:, :KTOP], axis=1, keepdims=True)
    sumv = jnp.sum(v8[:, :KTOP], axis=1, keepdims=True)
    revs = jnp.sum(cp_ref[...], axis=1, keepdims=True)
    s = jnp.maximum(1.0 + revs + 2.0 * sumw - sumv, 1e-8)
    y6 = y6_ref[...]
    y = jnp.concatenate([y6[q] for q in range(NSC * NF)], axis=1) / s
    emb = emb_ref[...]
    h = _gelu_exact(matT(y, wg_ref) + bg_ref[...]) + emb
    h1 = _gelu_exact(matT(h, w1_ref) + b1_ref[...])
    h2 = _gelu_exact(matT(h1, w2_ref) + b2_ref[...])
    r = matT(h2, wh_ref) + bh_ref[...]
    sp = jax.nn.softplus(r)
    col = lax.broadcasted_iota(jnp.int32, r.shape, 1)
    out8 = jnp.where(col == 0, r,
                     jnp.where(col == 2, jnp.minimum(sp, 28.0) + 1.01,
                               sp + 1e-6))
    out_ref[...] = out8[:, :4]


def _run_tail(out6, w8, v8, cpt, emb, Wg, bg2, W1, b12, W2, b22, Whp, bh2):
    whole = lambda shape: pl.BlockSpec(shape, lambda i: tuple(0 for _ in shape))
    return pl.pallas_call(
        _tail_body,
        grid=(NN // RB,),
        in_specs=[
            pl.BlockSpec((NSC * NF, RB, FC), lambda i: (0, i, 0)),
            pl.BlockSpec((RB, 8), lambda i: (i, 0)),
            pl.BlockSpec((RB, 8), lambda i: (i, 0)),
            pl.BlockSpec((RB, NN // RB), lambda i: (i, 0)),
            pl.BlockSpec((RB, DD), lambda i: (i, 0)),
            whole((DD, DD)), whole((1, DD)),
            whole((512, DD)), whole((1, 512)),
            whole((256, 512)), whole((1, 256)),
            whole((8, 256)), whole((1, 8)),
        ],
        out_specs=pl.BlockSpec((RB, 4), lambda i: (i, 0)),
        out_shape=jax.ShapeDtypeStruct((NN, 4), jnp.float32),
    )(out6, w8, v8, cpt, emb, Wg, bg2, W1, b12, W2, b22, Whp, bh2)


def kernel(emb, A, W_gcn, b_gcn, W1, b1, W2, b2, Wh, bh):
    ti8, v8, colpart, table6 = _run_topk(A, emb)
    tvec = v8[:, KTOP - 1]
    cvec = ti8[:, KTOP]
    w8 = _run_scw(ti8, v8, tvec, cvec)
    cpt = colpart.reshape(NN // RB, NN).T
    table = table6.reshape(NSC * NF * NN, FC)

    rows = jnp.arange(NN, dtype=jnp.int32)[:, None]
    ti5 = ti8[:, :KTOP]
    base6 = jnp.concatenate([rows, ti5], axis=1).reshape(-1)
    dlist = base6.reshape(NTILE, GCH, GW)
    qoff = (jnp.arange(NSC * NF, dtype=jnp.int32) * NN).reshape(
        NSC, NF, 1, 1, 1)
    glist = base6.reshape(1, 1, NTILE, GCH, GW) + qoff
    zrows = jnp.zeros((TROWS, FC), jnp.float32)

    out6 = _run_sc(glist, dlist, w8, table, zrows)

    Whp = jnp.pad(Wh, ((0, 4), (0, 0)))
    bh2 = jnp.pad(bh, (0, 4)).reshape(1, 8)
    return _run_tail(out6, w8, v8, cpt, emb, W_gcn, b_gcn.reshape(1, DD),
                     W1, b1.reshape(1, 512), W2, b2.reshape(1, 256), Whp,
                     bh2)
